# baseline (speedup 1.0000x reference)
import functools

import jax
import jax.numpy as jnp
from jax import lax
from jax.experimental import pallas as pl
from jax.experimental.pallas import tpu as pltpu
from jax.experimental.pallas import tpu_sc as plsc

EMBED_DIM = 64
NC = 2
NS = 16
NW = NC * NS
ROWS, SEQ = 4096, 50
B_PER_W = ROWS // NW
LANES = 16
DB = EMBED_DIM // 8
BB = ROWS // 128

_mesh = plsc.VectorSubcoreMesh(core_axis_name="c", subcore_axis_name="s")


@functools.partial(
    pl.kernel,
    mesh=_mesh,
    out_type=jax.ShapeDtypeStruct((SEQ, DB, BB, 8, 128), jnp.float32),
    scratch_types=[
        pltpu.VMEM((B_PER_W, SEQ), jnp.int32),
        pltpu.VMEM((SEQ, B_PER_W), jnp.int32),
        pltpu.VMEM((2, B_PER_W, EMBED_DIM), jnp.float32),
        pltpu.VMEM((2, 1, DB, 1, 8, 128), jnp.float32),
        pltpu.SemaphoreType.DMA,
        pltpu.SemaphoreType.DMA,
        pltpu.SemaphoreType.DMA,
        pltpu.SemaphoreType.DMA,
    ],
    compiler_params=pltpu.CompilerParams(
        use_tc_tiling_on_sc=False, needs_layout_passes=False
    ),
)
def _embed(src_hbm, table_hbm, out_hbm, idx_v, idxt_v, rows_v, tbuf_v,
           gsem0, gsem1, osem0, osem1):
    wid = lax.axis_index("s") * NC + lax.axis_index("c")
    bbase = wid * B_PER_W

    pltpu.sync_copy(src_hbm.at[pl.ds(bbase, B_PER_W)], idx_v)

    iot = lax.iota(jnp.int32, 16)
    bidx = [iot + LANES * k for k in range(B_PER_W // LANES)]

    @pl.loop(0, SEQ)
    def _(s):
        sidx = iot * 0 + s
        for k in range(B_PER_W // LANES):
            v = plsc.load_gather(idx_v, [bidx[k], sidx])
            idxt_v[s, pl.ds(LANES * k, LANES)] = v

    gsems = (gsem0, gsem1)
    osems = (osem0, osem1)

    def gather_start(s, buf):
        pltpu.async_copy(table_hbm.at[idxt_v.at[s]], rows_v.at[buf], gsems[buf])

    def gather_wait(buf):
        pltpu.make_async_copy(
            table_hbm.at[pl.ds(0, B_PER_W)], rows_v.at[buf], gsems[buf]
        ).wait()

    def transpose(buf, tb):
        src = rows_v.at[buf]
        for dblk in range(DB):
            for dsub in range(8):
                didx = iot * 0 + (8 * dblk + dsub)
                for k in range(B_PER_W // LANES):
                    v = plsc.load_gather(src, [bidx[k], didx])
                    tbuf_v[tb, 0, dblk, 0, dsub, pl.ds(LANES * k, LANES)] = v

    def emit_start(s, tb):
        pltpu.async_copy(
            tbuf_v.at[tb],
            out_hbm.at[pl.ds(s, 1), pl.ds(0, DB), pl.ds(wid, 1)],
            osems[tb],
        )

    def emit_wait(tb):
        pltpu.make_async_copy(
            out_hbm.at[pl.ds(0, 1), pl.ds(0, DB), pl.ds(0, 1)],
            tbuf_v.at[tb],
            osems[tb],
        ).wait()

    def step(s, buf, guard):
        if guard is None:
            emit_wait(buf)
        else:
            @pl.when(guard)
            def _():
                emit_wait(buf)
        transpose(buf, buf)
        emit_start(s, buf)

    gather_start(0, 0)

    @pl.loop(0, SEQ - 2, step=2)
    def _(g):
        gather_start(g + 1, 1)
        gather_wait(0)
        step(g, 0, g >= 2)
        gather_start(g + 2, 0)
        gather_wait(1)
        step(g + 1, 1, g >= 2)

    gather_start(SEQ - 1, 1)
    gather_wait(0)
    step(SEQ - 2, 0, None)
    gather_wait(1)
    step(SEQ - 1, 1, None)

    emit_wait(0)
    emit_wait(1)


def kernel(src, table):
    out1 = _embed(src.astype(jnp.int32), table)
    return out1.transpose(2, 4, 0, 1, 3).reshape(ROWS, SEQ, EMBED_DIM)


# speedup vs baseline: 1.7901x; 1.7901x over previous
import functools

import jax
import jax.numpy as jnp
from jax import lax
from jax.experimental import pallas as pl
from jax.experimental.pallas import tpu as pltpu
from jax.experimental.pallas import tpu_sc as plsc

EMBED_DIM = 64
NC = 2
NS = 16
NW = NC * NS
ROWS, SEQ = 4096, 50
R_PER_W = ROWS // NW
G = 8
NGROUP = R_PER_W // G
NBUF = 4

_mesh = plsc.VectorSubcoreMesh(core_axis_name="c", subcore_axis_name="s")


@functools.partial(
    pl.kernel,
    mesh=_mesh,
    out_type=jax.ShapeDtypeStruct((ROWS, SEQ, EMBED_DIM), jnp.float32),
    scratch_types=[
        pltpu.VMEM((R_PER_W, SEQ), jnp.int32),
        pltpu.VMEM((NBUF, G, SEQ, EMBED_DIM), jnp.float32),
        pltpu.SemaphoreType.DMA,
        pltpu.SemaphoreType.DMA,
        pltpu.SemaphoreType.DMA,
        pltpu.SemaphoreType.DMA,
        pltpu.SemaphoreType.DMA,
        pltpu.SemaphoreType.DMA,
        pltpu.SemaphoreType.DMA,
        pltpu.SemaphoreType.DMA,
    ],
    compiler_params=pltpu.CompilerParams(use_tc_tiling_on_sc=False),
)
def _embed(src_hbm, table_hbm, out_hbm, idx_v, rows_v,
           g0, g1, g2, g3, e0, e1, e2, e3):
    wid = lax.axis_index("s") * NC + lax.axis_index("c")
    rbase = wid * R_PER_W

    pltpu.sync_copy(src_hbm.at[pl.ds(rbase, R_PER_W)], idx_v)

    gsems = (g0, g1, g2, g3)
    esems = (e0, e1, e2, e3)

    def group_start(c, buf):
        for j in range(G):
            pltpu.async_copy(
                table_hbm.at[idx_v.at[c * G + j]], rows_v.at[buf, j], gsems[buf]
            )

    def group_wait(buf):
        pltpu.make_async_copy(
            out_hbm.at[pl.ds(0, G)], rows_v.at[buf], gsems[buf]
        ).wait()

    def emit_start(c, buf):
        pltpu.async_copy(
            rows_v.at[buf], out_hbm.at[pl.ds(rbase + c * G, G)], esems[buf]
        )

    def emit_wait(buf):
        pltpu.make_async_copy(
            out_hbm.at[pl.ds(0, G)], rows_v.at[buf], esems[buf]
        ).wait()

    for b in range(NBUF - 1):
        group_start(b, b)

    @pl.loop(0, NGROUP, step=NBUF)
    def _(g):
        for b in range(NBUF):
            c = g + b
            nxt = c + NBUF - 1
            pbuf = (b + NBUF - 1) % NBUF
            if b == 0:
                @pl.when(g > 0)
                def _():
                    emit_wait(pbuf)
            else:
                emit_wait(pbuf)

            @pl.when(nxt < NGROUP)
            def _():
                group_start(nxt, pbuf)

            group_wait(b)
            emit_start(c, b)

    emit_wait((NGROUP - 1) % NBUF)


def kernel(src, table):
    return _embed(src.astype(jnp.int32), table)
